# baseline (speedup 1.0000x reference)
import functools

import jax
import jax.numpy as jnp
from jax import lax
from jax.experimental import pallas as pl
from jax.experimental.pallas import tpu as pltpu
from jax.experimental.pallas import tpu_sc as plsc

D = H = W = 192
VOL = D * H * W
N_LANES = 16
NC = 2
NS = 16
NW = NC * NS
B_PER = 160
NPAD = NW * B_PER
CHUNKS = B_PER // N_LANES
HALF = B_PER // 2


def _make_sc_kernel():
    mesh = plsc.VectorSubcoreMesh(core_axis_name="c", subcore_axis_name="s")

    @functools.partial(
        pl.kernel,
        mesh=mesh,
        out_type=jax.ShapeDtypeStruct((3 * NPAD,), jnp.float32),
        scratch_types=[
            pltpu.VMEM((3 * B_PER,), jnp.float32),
            pltpu.VMEM((3 * B_PER,), jnp.float32),
            pltpu.VMEM((3 * N_LANES,), jnp.float32),
            pltpu.VMEM((12 * HALF,), jnp.int32),
            pltpu.VMEM((12 * HALF,), jnp.float32),
            pltpu.VMEM((3 * B_PER,), jnp.float32),
            pltpu.SemaphoreType.DMA,
        ],
    )
    def sc_kernel(mov_hbm, fix_hbm, sp_hbm, field_hbm, out_hbm,
                  mbuf, fbuf, spbuf, idxbuf, gbuf, obuf, sem):
        wid = lax.axis_index("s") * NC + lax.axis_index("c")
        base = wid * B_PER

        for ch in range(3):
            pltpu.sync_copy(mov_hbm.at[pl.ds(ch * NPAD + base, B_PER)],
                            mbuf.at[pl.ds(ch * B_PER, B_PER)])
            pltpu.sync_copy(fix_hbm.at[pl.ds(ch * NPAD + base, B_PER)],
                            fbuf.at[pl.ds(ch * B_PER, B_PER)])
        pltpu.sync_copy(sp_hbm, spbuf)

        for i in range(CHUNKS):
            k = i // (CHUNKS // 2)
            col = (i % (CHUNKS // 2)) * N_LANES
            fidx = None
            cidx = None
            for ch in range(3):
                m = mbuf[pl.ds(ch * B_PER + i * N_LANES, N_LANES)]
                f_i = m.astype(jnp.int32)
                c_i = jnp.where(m > f_i.astype(jnp.float32), f_i + 1, f_i)
                fidx = f_i if fidx is None else fidx * D + f_i
                cidx = c_i if cidx is None else cidx * D + c_i
            for ch in range(3):
                off = ch * VOL
                idxbuf[pl.ds((2 * ch + k) * HALF + col, N_LANES)] = fidx + off
                idxbuf[pl.ds((6 + 2 * ch + k) * HALF + col, N_LANES)] = (
                    cidx + off)

        copies = []
        for r in range(12):
            copies.append(
                pltpu.async_copy(
                    field_hbm.at[idxbuf.at[pl.ds(r * HALF, HALF)]],
                    gbuf.at[pl.ds(r * HALF, HALF)], sem))
        for cp in copies:
            cp.wait()

        for i in range(CHUNKS):
            k = i // (CHUNKS // 2)
            col = (i % (CHUNKS // 2)) * N_LANES
            for ch in range(3):
                gf = gbuf[pl.ds((2 * ch + k) * HALF + col, N_LANES)]
                gc = gbuf[pl.ds((6 + 2 * ch + k) * HALF + col, N_LANES)]
                m = mbuf[pl.ds(ch * B_PER + i * N_LANES, N_LANES)]
                fx = fbuf[pl.ds(ch * B_PER + i * N_LANES, N_LANES)]
                sp = spbuf[pl.ds(ch * N_LANES, N_LANES)]
                obuf[pl.ds(ch * B_PER + i * N_LANES, N_LANES)] = (
                    (m + (gf + gc) * 0.5 - fx) * sp)

        for ch in range(3):
            pltpu.sync_copy(obuf.at[pl.ds(ch * B_PER, B_PER)],
                            out_hbm.at[pl.ds(ch * NPAD + base, B_PER)])

    return sc_kernel


_SC_KERNEL = _make_sc_kernel()


def kernel(fixed_landmarks, moving_landmarks, displacement_field,
           fixed_spacing, moving_spacing):
    n = moving_landmarks.shape[0]
    mt = jnp.zeros((3, NPAD), jnp.float32).at[:, :n].set(
        moving_landmarks.T).reshape(3 * NPAD)
    ft = jnp.zeros((3, NPAD), jnp.float32).at[:, :n].set(
        fixed_landmarks.T).reshape(3 * NPAD)
    spb = jnp.broadcast_to(
        moving_spacing.astype(jnp.float32)[:, None],
        (3, N_LANES)).reshape(3 * N_LANES)
    field_flat = displacement_field.reshape(3 * VOL)
    out_t = _SC_KERNEL(mt, ft, spb, field_flat)
    return out_t.reshape(3, NPAD)[:, :n].T
